# baseline (speedup 1.0000x reference)
import jax
import jax.numpy as jnp
from jax import lax
from jax.experimental import pallas as pl

C = 1000
R = 512


def _onehot_body(idx_ref, out_ref):
    idx = idx_ref[...]
    iot = lax.broadcasted_iota(jnp.int32, (R, C), 1)
    out_ref[...] = (idx == iot).astype(jnp.float32)


def kernel(inputs):
    B1, B2 = inputs.shape
    N = B1 * B2
    idx2 = inputs.reshape(N, 1).astype(jnp.int32)
    out = pl.pallas_call(
        _onehot_body,
        grid=(N // R,),
        in_specs=[pl.BlockSpec((R, 1), lambda i: (i, 0))],
        out_specs=pl.BlockSpec((R, C), lambda i: (i, 0)),
        out_shape=jax.ShapeDtypeStruct((N, C), jnp.float32),
    )(idx2)
    return out.reshape(B1, B2, C)


# speedup vs baseline: 1.5122x; 1.5122x over previous
import jax
import jax.numpy as jnp
from jax import lax
from jax.experimental import pallas as pl
from jax.experimental.pallas import tpu as pltpu

C = 1000
G = 16
K = 4


def _onehot_body(idx_ref, out_ref, *scratch):
    bufs = scratch[:K]
    sems = scratch[K:]
    i = pl.program_id(0)
    n = pl.num_programs(0)

    idx = idx_ref[...]
    iot = lax.broadcasted_iota(jnp.int32, idx.shape + (C,), idx.ndim)
    val = (idx[..., None] == iot).astype(jnp.float32)

    slot = lax.rem(i, K)
    for k in range(K):
        @pl.when(slot == k)
        def _(k=k):
            @pl.when(i >= K)
            def _():
                pltpu.make_async_copy(
                    bufs[k], out_ref.at[pl.ds((i - K) * G, G)], sems[k]
                ).wait()
            bufs[k][...] = val
            pltpu.make_async_copy(
                bufs[k], out_ref.at[pl.ds(i * G, G)], sems[k]
            ).start(priority=k % 2)

    @pl.when(i == n - 1)
    def _():
        for j in range(K):
            s = i - j
            for k in range(K):
                @pl.when(lax.rem(s, K) == k)
                def _(s=s, k=k):
                    pltpu.make_async_copy(
                        bufs[k], out_ref.at[pl.ds(s * G, G)], sems[k]
                    ).wait()


def kernel(inputs):
    B1, B2 = inputs.shape
    return pl.pallas_call(
        _onehot_body,
        grid=(B1 // G,),
        in_specs=[pl.BlockSpec((G, B2), lambda i: (i, 0))],
        out_specs=pl.BlockSpec(memory_space=pltpu.HBM),
        out_shape=jax.ShapeDtypeStruct((B1, B2, C), jnp.float32),
        scratch_shapes=(
            [pltpu.VMEM((G, B2, C), jnp.float32) for _ in range(K)]
            + [pltpu.SemaphoreType.DMA for _ in range(K)]
        ),
        compiler_params=pltpu.CompilerParams(
            dimension_semantics=("arbitrary",),
        ),
    )(inputs.astype(jnp.int32))
